# baseline (speedup 1.0000x reference)
import functools

import jax
import jax.numpy as jnp
from jax import lax
from jax.experimental import pallas as pl
from jax.experimental.pallas import tpu as pltpu
from jax.experimental.pallas import tpu_sc as plsc

_NC = 2
_NS = 16
_NW = _NC * _NS

_LEV = (50000, 12500, 3125, 782, 196)
_OUTC = (32, 32, 32, 64)
_POOL_BLK = 128


def _ceil_to(x, m):
    return (x + m - 1) // m * m


def _sc_gather(table, idx, chunk):
    v_rows, d = table.shape
    m = idx.shape[0]
    m_pad = _ceil_to(m, _NW * chunk)
    if m_pad != m:
        idx = jnp.concatenate([idx, jnp.zeros((m_pad - m,), jnp.int32)])
    per_w = m_pad // _NW
    n_chunks = per_w // chunk

    mesh = plsc.VectorSubcoreMesh(core_axis_name="c", subcore_axis_name="s")

    @functools.partial(
        pl.kernel,
        mesh=mesh,
        compiler_params=pltpu.CompilerParams(use_tc_tiling_on_sc=False),
        out_type=jax.ShapeDtypeStruct((m_pad, d), table.dtype),
        scratch_types=[
            pltpu.VMEM((chunk,), jnp.int32),
            pltpu.VMEM((chunk, d), table.dtype),
            pltpu.SemaphoreType.DMA,
        ],
    )
    def gather_kernel(table_hbm, idx_hbm, out_hbm, idx_v, rows_v, sem):
        wid = lax.axis_index("s") * _NC + lax.axis_index("c")
        base = wid * per_w

        def body(i, _):
            off = base + i * chunk
            pltpu.sync_copy(idx_hbm.at[pl.ds(off, chunk)], idx_v)
            pltpu.async_copy(table_hbm.at[idx_v], rows_v, sem).wait()
            pltpu.sync_copy(rows_v, out_hbm.at[pl.ds(off, chunk)])
            return 0

        lax.fori_loop(0, n_chunks, body, 0)

    return gather_kernel(table, idx)


def _conv_kernel(g_ref, w_ref, b_ref, o_ref):
    acc = jnp.zeros(o_ref.shape, jnp.float32)
    for s in range(9):
        acc = acc + jnp.dot(g_ref[s], w_ref[s],
                            preferred_element_type=jnp.float32)
    acc = acc + b_ref[...]
    acc = jnp.where(acc > 0.0, acc, jnp.exp(jnp.minimum(acc, 0.0)) - 1.0)
    o_ref[...] = acc.astype(o_ref.dtype)


def _conv(g3, wbd, b_bd, vb, out_dtype):
    _, n, bc = g3.shape
    bo = wbd.shape[2]
    grid = (n + vb - 1) // vb
    return pl.pallas_call(
        _conv_kernel,
        grid=(grid,),
        in_specs=[
            pl.BlockSpec((9, vb, bc), lambda i: (0, i, 0)),
            pl.BlockSpec((9, bc, bo), lambda i: (0, 0, 0)),
            pl.BlockSpec((1, bo), lambda i: (0, 0)),
        ],
        out_specs=pl.BlockSpec((vb, bo), lambda i: (i, 0)),
        out_shape=jax.ShapeDtypeStruct((n, bo), out_dtype),
    )(g3, wbd, b_bd)


def _block_diag(w, batch):
    o, fan = w.shape
    c = fan // 9
    wp = w.reshape(o, 9, c).transpose(1, 2, 0)
    eye = jnp.eye(batch, dtype=w.dtype)
    wbd = jnp.einsum("sco,bd->sbcdo", wp, eye)
    return wbd.reshape(9, batch * c, batch * o)


def _mm_kernel(x_ref, w_ref, b_ref, o_ref, *, elu):
    acc = jnp.dot(x_ref[...], w_ref[...], preferred_element_type=jnp.float32)
    acc = acc + b_ref[...]
    if elu:
        acc = jnp.where(acc > 0.0, acc, jnp.exp(jnp.minimum(acc, 0.0)) - 1.0)
    o_ref[...] = acc


def _matmul(x, w_t, b, elu, mb=2048):
    m, k = x.shape
    o = w_t.shape[1]
    grid = (m + mb - 1) // mb
    return pl.pallas_call(
        functools.partial(_mm_kernel, elu=elu),
        grid=(grid,),
        in_specs=[
            pl.BlockSpec((mb, k), lambda i: (i, 0)),
            pl.BlockSpec((k, o), lambda i: (0, 0)),
            pl.BlockSpec((1, o), lambda i: (0, 0)),
        ],
        out_specs=pl.BlockSpec((mb, o), lambda i: (i, 0)),
        out_shape=jax.ShapeDtypeStruct((m, o), jnp.float32),
    )(x, w_t, b[None, :])


def _pool_kernel(g_ref, d_ref, o_ref):
    nb = o_ref.shape[0]
    r = lax.broadcasted_iota(jnp.int32, (nb, 3 * nb), 0)
    c = lax.broadcasted_iota(jnp.int32, (nb, 3 * nb), 1)
    p = jnp.zeros((nb, 3 * nb), jnp.float32)
    for t in range(3):
        p = p + jnp.where(c == 3 * r + t, d_ref[:, t : t + 1], 0.0)
    acc = jnp.dot(p.astype(g_ref.dtype), g_ref[...],
                  preferred_element_type=jnp.float32)
    o_ref[...] = acc.astype(o_ref.dtype)


def _pool(g_pad, d_pad, d_cols, grid, out_dtype):
    return pl.pallas_call(
        _pool_kernel,
        grid=(grid,),
        in_specs=[
            pl.BlockSpec((3 * _POOL_BLK, d_cols), lambda i: (i, 0)),
            pl.BlockSpec((_POOL_BLK, 3), lambda i: (i, 0)),
        ],
        out_specs=pl.BlockSpec((_POOL_BLK, d_cols), lambda i: (i, 0)),
        out_shape=jax.ShapeDtypeStruct((grid * _POOL_BLK, d_cols), out_dtype),
    )(g_pad, d_pad)


def kernel(x, spiral_0, spiral_1, spiral_2, spiral_3, row_0, row_1, row_2,
           row_3, col_0, col_1, col_2, col_3, data_0, data_1, data_2, data_3,
           W_0, b_0, W_1, b_1, W_2, b_2, W_3, b_3, W_fc, b_fc):
    spirals = (spiral_0, spiral_1, spiral_2, spiral_3)
    cols = (col_0, col_1, col_2, col_3)
    datas = (data_0, data_1, data_2, data_3)
    ws = (W_0, W_1, W_2, W_3)
    bs = (b_0, b_1, b_2, b_3)

    batch = x.shape[0]
    c_in = x.shape[2]
    h = x.transpose(1, 0, 2).reshape(_LEV[0], batch * c_in)

    for i in range(4):
        n, n_out, o = _LEV[i], _LEV[i + 1], _OUTC[i]
        d_in = batch * c_in
        ch = 128 if d_in * h.dtype.itemsize <= 1024 else 64

        g = _sc_gather(h, spirals[i].T.reshape(-1), ch)[: 9 * n]
        g3 = g.reshape(9, n, d_in)
        wbd = _block_diag(ws[i], batch).astype(h.dtype)
        b_bd = jnp.tile(bs[i], batch)[None, :]
        vb = 128 if batch * o > 512 else 256
        h2 = _conv(g3, wbd, b_bd, vb, jnp.bfloat16)

        d_out = batch * o
        ch2 = 128 if d_out * 2 <= 1024 else 64
        gp = _sc_gather(h2, cols[i], ch2)
        grid = (n_out + _POOL_BLK - 1) // _POOL_BLK
        assert gp.shape[0] >= 3 * _POOL_BLK * grid
        d_mat = datas[i].reshape(n_out, 3)
        d_mat = jnp.concatenate(
            [d_mat, jnp.zeros((grid * _POOL_BLK - n_out, 3), jnp.float32)])
        h = _pool(gp, d_mat, d_out, grid, jnp.bfloat16)
        c_in = o

    hf = h[: _LEV[4]].reshape(_LEV[4], batch, _OUTC[3]).transpose(1, 0, 2)
    hf = hf.reshape(batch, _LEV[4] * _OUTC[3])
    return _matmul(hf, W_fc.T.astype(hf.dtype), b_fc, elu=False, mb=32)


# speedup vs baseline: 1.1778x; 1.1778x over previous
import functools

import jax
import jax.numpy as jnp
from jax import lax
from jax.experimental import pallas as pl
from jax.experimental.pallas import tpu as pltpu
from jax.experimental.pallas import tpu_sc as plsc

_NC = 2
_NS = 16
_NW = _NC * _NS

_LEV = (50000, 12500, 3125, 782, 196)
_OUTC = (32, 32, 32, 64)
_POOL_BLK = 128


def _ceil_to(x, m):
    return (x + m - 1) // m * m


def _sc_gather(table, idx, chunk):
    v_rows, d = table.shape
    m = idx.shape[0]
    m_pad = _ceil_to(m, _NW * chunk)
    if m_pad != m:
        idx = jnp.concatenate([idx, jnp.zeros((m_pad - m,), jnp.int32)])
    per_w = m_pad // _NW
    n_chunks = per_w // chunk

    mesh = plsc.VectorSubcoreMesh(core_axis_name="c", subcore_axis_name="s")

    @functools.partial(
        pl.kernel,
        mesh=mesh,
        compiler_params=pltpu.CompilerParams(use_tc_tiling_on_sc=False),
        out_type=jax.ShapeDtypeStruct((m_pad, d), table.dtype),
        scratch_types=[
            pltpu.VMEM((chunk,), jnp.int32),
            pltpu.VMEM((chunk, d), table.dtype),
            pltpu.SemaphoreType.DMA,
        ],
    )
    def gather_kernel(table_hbm, idx_hbm, out_hbm, idx_v, rows_v, sem):
        wid = lax.axis_index("s") * _NC + lax.axis_index("c")
        base = wid * per_w

        def body(i, _):
            off = base + i * chunk
            pltpu.sync_copy(idx_hbm.at[pl.ds(off, chunk)], idx_v)
            pltpu.async_copy(table_hbm.at[idx_v], rows_v, sem).wait()
            pltpu.sync_copy(rows_v, out_hbm.at[pl.ds(off, chunk)])
            return 0

        lax.fori_loop(0, n_chunks, body, 0)

    return gather_kernel(table, idx)


def _conv_kernel(g_ref, w_ref, b_ref, o_ref):
    acc = jnp.zeros(o_ref.shape, jnp.float32)
    for s in range(9):
        acc = acc + jnp.dot(g_ref[s], w_ref[s],
                            preferred_element_type=jnp.float32)
    acc = acc + b_ref[...]
    acc = jnp.where(acc > 0.0, acc, jnp.exp(jnp.minimum(acc, 0.0)) - 1.0)
    o_ref[...] = acc.astype(o_ref.dtype)


def _conv(g3, wbd, b_bd, vb, out_dtype):
    _, n, bc = g3.shape
    bo = wbd.shape[2]
    grid = (n + vb - 1) // vb
    return pl.pallas_call(
        _conv_kernel,
        grid=(grid,),
        in_specs=[
            pl.BlockSpec((9, vb, bc), lambda i: (0, i, 0)),
            pl.BlockSpec((9, bc, bo), lambda i: (0, 0, 0)),
            pl.BlockSpec((1, bo), lambda i: (0, 0)),
        ],
        out_specs=pl.BlockSpec((vb, bo), lambda i: (i, 0)),
        out_shape=jax.ShapeDtypeStruct((n, bo), out_dtype),
    )(g3, wbd, b_bd)


def _block_diag(w, batch):
    o, fan = w.shape
    c = fan // 9
    wp = w.reshape(o, 9, c).transpose(1, 2, 0)
    eye = jnp.eye(batch, dtype=w.dtype)
    wbd = jnp.einsum("sco,bd->sbcdo", wp, eye)
    return wbd.reshape(9, batch * c, batch * o)


def _mm_kernel(x_ref, w_ref, b_ref, o_ref, *, elu):
    acc = jnp.dot(x_ref[...], w_ref[...], preferred_element_type=jnp.float32)
    acc = acc + b_ref[...]
    if elu:
        acc = jnp.where(acc > 0.0, acc, jnp.exp(jnp.minimum(acc, 0.0)) - 1.0)
    o_ref[...] = acc


def _matmul(x, w_t, b, elu, mb=2048):
    m, k = x.shape
    o = w_t.shape[1]
    grid = (m + mb - 1) // mb
    return pl.pallas_call(
        functools.partial(_mm_kernel, elu=elu),
        grid=(grid,),
        in_specs=[
            pl.BlockSpec((mb, k), lambda i: (i, 0)),
            pl.BlockSpec((k, o), lambda i: (0, 0)),
            pl.BlockSpec((1, o), lambda i: (0, 0)),
        ],
        out_specs=pl.BlockSpec((mb, o), lambda i: (i, 0)),
        out_shape=jax.ShapeDtypeStruct((m, o), jnp.float32),
    )(x, w_t, b[None, :])


def _pool_kernel(g_ref, d_ref, o_ref):
    nb = o_ref.shape[0]
    r = lax.broadcasted_iota(jnp.int32, (nb, 3 * nb), 0)
    c = lax.broadcasted_iota(jnp.int32, (nb, 3 * nb), 1)
    p = jnp.zeros((nb, 3 * nb), jnp.float32)
    for t in range(3):
        p = p + jnp.where(c == 3 * r + t, d_ref[:, t : t + 1], 0.0)
    acc = jnp.dot(p.astype(g_ref.dtype), g_ref[...],
                  preferred_element_type=jnp.float32)
    o_ref[...] = acc.astype(o_ref.dtype)


def _pool(g_pad, d_pad, d_cols, grid, out_dtype):
    return pl.pallas_call(
        _pool_kernel,
        grid=(grid,),
        in_specs=[
            pl.BlockSpec((3 * _POOL_BLK, d_cols), lambda i: (i, 0)),
            pl.BlockSpec((_POOL_BLK, 3), lambda i: (i, 0)),
        ],
        out_specs=pl.BlockSpec((_POOL_BLK, d_cols), lambda i: (i, 0)),
        out_shape=jax.ShapeDtypeStruct((grid * _POOL_BLK, d_cols), out_dtype),
    )(g_pad, d_pad)


def kernel(x, spiral_0, spiral_1, spiral_2, spiral_3, row_0, row_1, row_2,
           row_3, col_0, col_1, col_2, col_3, data_0, data_1, data_2, data_3,
           W_0, b_0, W_1, b_1, W_2, b_2, W_3, b_3, W_fc, b_fc):
    spirals = (spiral_0, spiral_1, spiral_2, spiral_3)
    cols = (col_0, col_1, col_2, col_3)
    datas = (data_0, data_1, data_2, data_3)
    ws = (W_0, W_1, W_2, W_3)
    bs = (b_0, b_1, b_2, b_3)

    batch = x.shape[0]
    c_in = x.shape[2]
    h = x.transpose(1, 0, 2).reshape(_LEV[0], batch * c_in)

    for i in range(4):
        n, n_out, o = _LEV[i], _LEV[i + 1], _OUTC[i]
        d_in = batch * c_in
        ch = 128 if d_in <= 128 else (64 if d_in <= 512 else 32)

        g = _sc_gather(h, spirals[i].T.reshape(-1), ch)[: 9 * n]
        g3 = g.reshape(9, n, d_in)
        wbd = _block_diag(ws[i], batch).astype(h.dtype)
        b_bd = jnp.tile(bs[i], batch)[None, :]
        vb = 128 if batch * o > 512 else 256
        h2 = _conv(g3, wbd, b_bd, vb, jnp.float32)

        d_out = batch * o
        ch2 = 64 if d_out <= 512 else 32
        gp = _sc_gather(h2, cols[i], ch2)
        grid = (n_out + _POOL_BLK - 1) // _POOL_BLK
        assert gp.shape[0] >= 3 * _POOL_BLK * grid
        d_mat = datas[i].reshape(n_out, 3)
        d_mat = jnp.concatenate(
            [d_mat, jnp.zeros((grid * _POOL_BLK - n_out, 3), jnp.float32)])
        h = _pool(gp, d_mat, d_out, grid, jnp.float32)
        c_in = o

    hf = h[: _LEV[4]].reshape(_LEV[4], batch, _OUTC[3]).transpose(1, 0, 2)
    hf = hf.reshape(batch, _LEV[4] * _OUTC[3])
    return _matmul(hf, W_fc.T.astype(hf.dtype), b_fc, elu=False, mb=32)
